# baseline (speedup 1.0000x reference)
import functools
import math

import jax
import jax.numpy as jnp
from jax import lax
from jax.experimental import pallas as pl
from jax.experimental.pallas import tpu as pltpu
from jax.experimental.pallas import tpu_sc as plsc

D_MODEL = 64
NUM_CORES = 2
NUM_SUBCORES = 16
NUM_WORKERS = NUM_CORES * NUM_SUBCORES
LANES = 16
S_HALF = 4
ITEM = S_HALF * 128
T_W = 129
SCALE = math.sqrt(D_MODEL)


def kernel(x, table):
    b_dim, s_dim = x.shape
    batch = b_dim * s_dim
    n_items = batch // ITEM
    per_worker = n_items // NUM_WORKERS
    bt = b_dim // 128
    st = s_dim // 8

    xv = (
        x.reshape(bt, 128, st, 8)
        .transpose(2, 0, 3, 1)
        .reshape(batch)
        .astype(jnp.int32)
    )

    mesh = plsc.VectorSubcoreMesh(core_axis_name="c", subcore_axis_name="s")

    @functools.partial(
        pl.kernel,
        mesh=mesh,
        out_type=jax.ShapeDtypeStruct((s_dim * 8 * bt, 8, 128), jnp.float32),
        compiler_params=pltpu.CompilerParams(
            use_tc_tiling_on_sc=False, needs_layout_passes=False
        ),
        scratch_types=[
            pltpu.VMEM((2 * ITEM,), jnp.int32),
            pltpu.VMEM((2 * ITEM, D_MODEL), jnp.float32),
            pltpu.VMEM((2 * D_MODEL, T_W), jnp.float32),
            pltpu.SemaphoreType.DMA((2,)),
            pltpu.SemaphoreType.DMA((2,)),
            pltpu.SemaphoreType.DMA((2,)),
        ],
    )
    def gather_t(table_hbm, idx_hbm, out_hbm, idx_v, g_v, t_v, isem, gsem, ssem):
        wid = lax.axis_index("s") * NUM_CORES + lax.axis_index("c")
        item0 = wid * per_worker

        def idx_dma(i, b):
            src = idx_hbm.at[pl.ds((item0 + i) * ITEM, ITEM)]
            return pltpu.make_async_copy(src, idx_v.at[pl.ds(b * ITEM, ITEM)], isem.at[b])

        def gather(b):
            src = table_hbm.at[idx_v.at[pl.ds(b * ITEM, ITEM)]]
            return pltpu.make_async_copy(src, g_v.at[pl.ds(b * ITEM, ITEM)], gsem.at[b])

        def stores(i, si, tt):
            j = item0 + i
            s8 = j // (2 * bt)
            t = (j % (2 * bt)) // 2
            h = j % 2
            s = 8 * s8 + S_HALF * h + si
            cps = []
            for k in range(8):
                src = t_v.at[pl.ds(tt * D_MODEL + 8 * k, 8), pl.ds(0, 128)]
                dst = out_hbm.at[(s * 8 + k) * bt + t]
                cps.append(pltpu.make_async_copy(src, dst, ssem.at[tt]))
            return cps

        def transpose_scale(b, si, tt):
            @plsc.parallel_loop(0, 128, unroll=8)
            def _(bi):
                row = b * ITEM + si * 128 + bi
                col = jnp.full((LANES,), bi, jnp.int32)
                for d0 in range(0, D_MODEL, LANES):
                    v = g_v.at[row, pl.ds(d0, LANES)][...] * SCALE
                    didx = tt * D_MODEL + d0 + lax.iota(jnp.int32, LANES)
                    plsc.store_scatter(t_v, [didx, col], v)

        def run_item(i, b, guard_first):
            gather(b).wait()
            for si in range(S_HALF):
                tt = si % 2
                if guard_first and si < 2:
                    @pl.when(i > 0)
                    def _():
                        for cp in stores(i, si, tt):
                            cp.wait()
                else:
                    for cp in stores(i, si, tt):
                        cp.wait()
                transpose_scale(b, si, tt)
                for cp in stores(i, si, tt):
                    cp.start()
            nxt = jnp.minimum(i + 2, per_worker - 1)
            idx_dma(nxt, b).start()
            idx_dma(nxt, b).wait()
            gather(b).start()

        idx_dma(0, 0).start()
        idx_dma(1, 1).start()
        idx_dma(0, 0).wait()
        gather(0).start()
        idx_dma(1, 1).wait()
        gather(1).start()

        @pl.loop(0, per_worker, step=2)
        def _(i):
            run_item(i, 0, True)
            run_item(i + 1, 1, False)

        gather(0).wait()
        gather(1).wait()
        for tt in range(2):
            for cp in stores(per_worker - 1, 2 + tt, tt):
                cp.wait()

    out5 = gather_t(table, xv)
    out = (
        out5.reshape(s_dim, 8, bt, 8, 128)
        .transpose(2, 4, 0, 1, 3)
        .reshape(b_dim, s_dim, D_MODEL)
    )
    return out


# speedup vs baseline: 1.0453x; 1.0453x over previous
import functools
import math

import jax
import jax.numpy as jnp
from jax import lax
from jax.experimental import pallas as pl
from jax.experimental.pallas import tpu as pltpu
from jax.experimental.pallas import tpu_sc as plsc

D_MODEL = 64
NUM_CORES = 2
NUM_SUBCORES = 16
NUM_WORKERS = NUM_CORES * NUM_SUBCORES
LANES = 16
S_HALF = 4
ITEM = S_HALF * 128
T_W = 129
SCALE = math.sqrt(D_MODEL)


def kernel(x, table):
    b_dim, s_dim = x.shape
    batch = b_dim * s_dim
    n_items = batch // ITEM
    per_worker = n_items // NUM_WORKERS
    bt = b_dim // 128
    st = s_dim // 8

    xv = x.T.astype(jnp.int32)

    mesh = plsc.VectorSubcoreMesh(core_axis_name="c", subcore_axis_name="s")

    @functools.partial(
        pl.kernel,
        mesh=mesh,
        out_type=jax.ShapeDtypeStruct((s_dim * 8 * bt, 8, 128), jnp.float32),
        compiler_params=pltpu.CompilerParams(
            use_tc_tiling_on_sc=False, needs_layout_passes=False
        ),
        scratch_types=[
            pltpu.VMEM((2, 1, ITEM), jnp.int32),
            pltpu.VMEM((2 * ITEM,), jnp.int32),
            pltpu.VMEM((2 * ITEM, D_MODEL), jnp.float32),
            pltpu.VMEM((2 * D_MODEL, T_W), jnp.float32),
            pltpu.SemaphoreType.DMA((2,)),
            pltpu.SemaphoreType.DMA((2,)),
            pltpu.SemaphoreType.DMA((2,)),
        ],
    )
    def gather_t(table_hbm, idx_hbm, out_hbm, idx_v, idx1_v, g_v, t_v, isem, gsem, ssem):
        wid = lax.axis_index("s") * NUM_CORES + lax.axis_index("c")
        item0 = wid * per_worker

        def idx_dma(i, b):
            j = item0 + i
            s8 = j // (2 * bt)
            t = (j % (2 * bt)) // 2
            h = j % 2
            cps = []
            for si in range(S_HALF):
                src = idx_hbm.at[
                    pl.ds(8 * s8 + S_HALF * h + si, 1), pl.ds(t * 128, 128)
                ]
                dst = idx_v.at[b, :, pl.ds(si * 128, 128)]
                cps.append(pltpu.make_async_copy(src, dst, isem.at[b]))
            return cps

        def idx_stage(b):
            @plsc.parallel_loop(0, ITEM, step=LANES)
            def _(q):
                idx1_v.at[pl.ds(b * ITEM + q, LANES)][...] = idx_v.at[
                    b, 0, pl.ds(q, LANES)
                ][...]

        def gather(b):
            src = table_hbm.at[idx1_v.at[pl.ds(b * ITEM, ITEM)]]
            return pltpu.make_async_copy(src, g_v.at[pl.ds(b * ITEM, ITEM)], gsem.at[b])

        def stores(i, si, tt):
            j = item0 + i
            s8 = j // (2 * bt)
            t = (j % (2 * bt)) // 2
            h = j % 2
            s = 8 * s8 + S_HALF * h + si
            cps = []
            for k in range(8):
                src = t_v.at[pl.ds(tt * D_MODEL + 8 * k, 8), pl.ds(0, 128)]
                dst = out_hbm.at[(s * 8 + k) * bt + t]
                cps.append(pltpu.make_async_copy(src, dst, ssem.at[tt]))
            return cps

        def transpose_scale(b, si, tt):
            @plsc.parallel_loop(0, 128, unroll=4)
            def _(bi):
                row = b * ITEM + si * 128 + bi
                col = jnp.full((LANES,), bi, jnp.int32)
                for d0 in range(0, D_MODEL, LANES):
                    v = g_v.at[row, pl.ds(d0, LANES)][...] * SCALE
                    didx = tt * D_MODEL + d0 + lax.iota(jnp.int32, LANES)
                    plsc.store_scatter(t_v, [didx, col], v)

        def run_item(i, b, guard_first):
            gather(b).wait()
            for si in range(S_HALF):
                tt = si % 2
                if guard_first and si < 2:
                    @pl.when(i > 0)
                    def _():
                        for cp in stores(i, si, tt):
                            cp.wait()
                else:
                    for cp in stores(i, si, tt):
                        cp.wait()
                transpose_scale(b, si, tt)
                for cp in stores(i, si, tt):
                    cp.start()
            nxt = jnp.minimum(i + 2, per_worker - 1)
            for cp in idx_dma(nxt, b):
                cp.start()
            for cp in idx_dma(nxt, b):
                cp.wait()
            idx_stage(b)
            gather(b).start()

        for cp in idx_dma(0, 0):
            cp.start()
        for cp in idx_dma(1, 1):
            cp.start()
        for cp in idx_dma(0, 0):
            cp.wait()
        idx_stage(0)
        gather(0).start()
        for cp in idx_dma(1, 1):
            cp.wait()
        idx_stage(1)
        gather(1).start()

        @pl.loop(0, per_worker, step=2)
        def _(i):
            run_item(i, 0, True)
            run_item(i + 1, 1, False)

        gather(0).wait()
        gather(1).wait()
        for tt in range(2):
            for cp in stores(per_worker - 1, 2 + tt, tt):
                cp.wait()

    out5 = gather_t(table, xv)
    out = (
        out5.reshape(s_dim, 8, bt, 8, 128)
        .transpose(2, 4, 0, 1, 3)
        .reshape(b_dim, s_dim, D_MODEL)
    )
    return out


# speedup vs baseline: 1.0571x; 1.0113x over previous
import functools
import math

import jax
import jax.numpy as jnp
from jax import lax
from jax.experimental import pallas as pl
from jax.experimental.pallas import tpu as pltpu
from jax.experimental.pallas import tpu_sc as plsc

D_MODEL = 64
NUM_CORES = 2
NUM_SUBCORES = 16
NUM_WORKERS = NUM_CORES * NUM_SUBCORES
LANES = 16
S_HALF = 4
ITEM = S_HALF * 128
T_W = 129
SCALE = math.sqrt(D_MODEL)


def kernel(x, table):
    b_dim, s_dim = x.shape
    batch = b_dim * s_dim
    n_items = batch // ITEM
    per_worker = n_items // NUM_WORKERS
    bt = b_dim // 128
    st = s_dim // 8

    xv = (
        x.reshape(bt, 128, st, 8)
        .transpose(2, 0, 3, 1)
        .reshape(batch)
        .astype(jnp.int32)
    )

    mesh = plsc.VectorSubcoreMesh(core_axis_name="c", subcore_axis_name="s")

    @functools.partial(
        pl.kernel,
        mesh=mesh,
        out_type=jax.ShapeDtypeStruct((s_dim * 8 * bt, 8, 128), jnp.float32),
        compiler_params=pltpu.CompilerParams(
            use_tc_tiling_on_sc=False, needs_layout_passes=False
        ),
        scratch_types=[
            pltpu.VMEM((2 * ITEM,), jnp.int32),
            pltpu.VMEM((2 * ITEM, D_MODEL), jnp.float32),
            pltpu.VMEM((2 * D_MODEL, T_W), jnp.float32),
            pltpu.SemaphoreType.DMA((2,)),
            pltpu.SemaphoreType.DMA((2,)),
            pltpu.SemaphoreType.DMA((2,)),
        ],
    )
    def gather_t(table_hbm, idx_hbm, out_hbm, idx_v, g_v, t_v, isem, gsem, ssem):
        wid = lax.axis_index("s") * NUM_CORES + lax.axis_index("c")
        item0 = wid * per_worker

        def idx_dma(i, b):
            src = idx_hbm.at[pl.ds((item0 + i) * ITEM, ITEM)]
            return [
                pltpu.make_async_copy(
                    src, idx_v.at[pl.ds(b * ITEM, ITEM)], isem.at[b]
                )
            ]

        def gather(b):
            src = table_hbm.at[idx_v.at[pl.ds(b * ITEM, ITEM)]]
            return pltpu.make_async_copy(src, g_v.at[pl.ds(b * ITEM, ITEM)], gsem.at[b])

        def stores(i, si, tt):
            j = item0 + i
            s8 = j // (2 * bt)
            t = (j % (2 * bt)) // 2
            h = j % 2
            s = 8 * s8 + S_HALF * h + si
            cps = []
            for k in range(8):
                src = t_v.at[pl.ds(tt * D_MODEL + 8 * k, 8), pl.ds(0, 128)]
                dst = out_hbm.at[(s * 8 + k) * bt + t]
                cps.append(pltpu.make_async_copy(src, dst, ssem.at[tt]))
            return cps

        def transpose_scale(b, si, tt):
            @plsc.parallel_loop(0, 128, unroll=4)
            def _(bi):
                row = b * ITEM + si * 128 + bi
                col = jnp.full((LANES,), bi, jnp.int32)
                for d0 in range(0, D_MODEL, LANES):
                    v = g_v.at[row, pl.ds(d0, LANES)][...] * SCALE
                    didx = tt * D_MODEL + d0 + lax.iota(jnp.int32, LANES)
                    plsc.store_scatter(t_v, [didx, col], v)

        def run_item(i, b, guard_first):
            gather(b).wait()
            for si in range(S_HALF):
                tt = si % 2
                if guard_first and si < 2:
                    @pl.when(i > 0)
                    def _():
                        for cp in stores(i, si, tt):
                            cp.wait()
                else:
                    for cp in stores(i, si, tt):
                        cp.wait()
                transpose_scale(b, si, tt)
                for cp in stores(i, si, tt):
                    cp.start()
            nxt = jnp.minimum(i + 2, per_worker - 1)
            for cp in idx_dma(nxt, b):
                cp.start()
            for cp in idx_dma(nxt, b):
                cp.wait()
            gather(b).start()

        for cp in idx_dma(0, 0):
            cp.start()
        for cp in idx_dma(1, 1):
            cp.start()
        for cp in idx_dma(0, 0):
            cp.wait()
        gather(0).start()
        for cp in idx_dma(1, 1):
            cp.wait()
        gather(1).start()

        @pl.loop(0, per_worker, step=2)
        def _(i):
            run_item(i, 0, True)
            run_item(i + 1, 1, False)

        gather(0).wait()
        gather(1).wait()
        for tt in range(2):
            for cp in stores(per_worker - 1, 2 + tt, tt):
                cp.wait()

    out5 = gather_t(table, xv)
    out = (
        out5.reshape(s_dim, 8, bt, 8, 128)
        .transpose(2, 4, 0, 1, 3)
        .reshape(b_dim, s_dim, D_MODEL)
    )
    return out
